# baseline (speedup 1.0000x reference)
import math

import jax
import jax.numpy as jnp
from jax.experimental import pallas as pl
from jax.experimental.pallas import tpu as pltpu

D_MODEL = 2048
NUM_LVL = 3
SEG = 16
S_TOTAL = NUM_LVL * SEG
INV_SQRT_D = 1.0 / math.sqrt(D_MODEL)
LEVEL_W = 1.0 / NUM_LVL
CHUNK = 256
N_CHUNKS = 64
NBUF = 6
HALVES = 2


def _compute_chunk(qbuf_slot, kt_ref, v_ref, seg_ref, obuf_slot):
    h = CHUNK // HALVES
    for p in range(HALVES):
        rows = pl.ds(p * h, h)
        q = qbuf_slot[rows, :].astype(jnp.bfloat16)
        s = jnp.dot(q, kt_ref[...], preferred_element_type=jnp.float32)
        e = jnp.exp(s)
        z = jax.lax.dot_general(
            e, seg_ref[...], (((1,), (0,)), ((), ())),
            precision=jax.lax.Precision.HIGHEST,
            preferred_element_type=jnp.float32,
        )
        a = (e * (1.0 / z)).astype(jnp.bfloat16)
        obuf_slot[rows, :] = jnp.dot(a, v_ref[...],
                                     preferred_element_type=jnp.float32)


def _attn_kernel(q_hbm, kt_ref, v_ref, seg_ref, o_hbm,
                 qbuf, obuf, in_sems, out_sems):
    def in_copy(i, slot):
        return pltpu.make_async_copy(
            q_hbm.at[pl.ds(i * CHUNK, CHUNK), :], qbuf.at[slot],
            in_sems.at[slot])

    def out_copy(i, slot):
        return pltpu.make_async_copy(
            obuf.at[slot], o_hbm.at[pl.ds(i * CHUNK, CHUNK), :],
            out_sems.at[slot])

    for s in range(NBUF):
        in_copy(s, s).start()

    def loop(i, carry):
        slot = jax.lax.rem(i, NBUF)
        in_copy(i, slot).wait()

        @pl.when(i >= NBUF)
        def _():
            out_copy(i - NBUF, slot).wait()

        _compute_chunk(qbuf.at[slot], kt_ref, v_ref, seg_ref, obuf.at[slot])
        out_copy(i, slot).start()

        @pl.when(i + NBUF < N_CHUNKS)
        def _():
            in_copy(i + NBUF, slot).start()

        return carry

    jax.lax.fori_loop(0, N_CHUNKS, loop, 0)
    for s in range(NBUF):
        i = N_CHUNKS - NBUF + s
        out_copy(i, i % NBUF).wait()


@jax.jit
def kernel(query, keys_0, values_0, salience_0, keys_1, values_1, salience_1,
           keys_2, values_2, salience_2):
    B, T, D = query.shape
    q2 = query.reshape(B * T, D)
    kt = (jnp.concatenate([keys_0, keys_1, keys_2], axis=0).T
          * INV_SQRT_D).astype(jnp.bfloat16)
    w = jnp.exp(jnp.concatenate([salience_0, salience_1, salience_2]))
    v = (jnp.concatenate([values_0, values_1, values_2], axis=0)
         * (w[:, None] * LEVEL_W)).astype(jnp.bfloat16)
    lvl = jnp.arange(S_TOTAL) // SEG
    seg = (lvl[:, None] == lvl[None, :]).astype(jnp.float32) * w[:, None]
    out = pl.pallas_call(
        _attn_kernel,
        in_specs=[
            pl.BlockSpec(memory_space=pltpu.MemorySpace.HBM),
            pl.BlockSpec(memory_space=pltpu.MemorySpace.VMEM),
            pl.BlockSpec(memory_space=pltpu.MemorySpace.VMEM),
            pl.BlockSpec(memory_space=pltpu.MemorySpace.VMEM),
        ],
        out_specs=pl.BlockSpec(memory_space=pltpu.MemorySpace.HBM),
        out_shape=jax.ShapeDtypeStruct((B * T, D), jnp.float32),
        scratch_shapes=[
            pltpu.VMEM((NBUF, CHUNK, D_MODEL), jnp.float32),
            pltpu.VMEM((NBUF, CHUNK, D_MODEL), jnp.float32),
            pltpu.SemaphoreType.DMA((NBUF,)),
            pltpu.SemaphoreType.DMA((NBUF,)),
        ],
    )(q2, kt, v, seg)
    return out.reshape(B, T, D)


# speedup vs baseline: 1.1876x; 1.1876x over previous
import math

import jax
import jax.numpy as jnp
from jax.experimental import pallas as pl
from jax.experimental.pallas import tpu as pltpu

D_MODEL = 2048
NUM_LVL = 3
SEG = 16
S_TOTAL = NUM_LVL * SEG
INV_SQRT_D = 1.0 / math.sqrt(D_MODEL)
LEVEL_W = 1.0 / NUM_LVL
CHUNK = 1024
N_CHUNKS = 16
NBUF = 3
HALVES = 2


def _compute_chunk(qbuf_slot, kt_ref, v_ref, seg_ref, obuf_slot):
    h = CHUNK // HALVES
    for p in range(HALVES):
        rows = pl.ds(p * h, h)
        q = qbuf_slot[rows, :].astype(jnp.bfloat16)
        s = jnp.dot(q, kt_ref[...], preferred_element_type=jnp.float32)
        e = jnp.exp(s)
        z = jax.lax.dot_general(
            e, seg_ref[...], (((1,), (0,)), ((), ())),
            precision=jax.lax.Precision.HIGHEST,
            preferred_element_type=jnp.float32,
        )
        a = (e * (1.0 / z)).astype(jnp.bfloat16)
        obuf_slot[rows, :] = jnp.dot(a, v_ref[...],
                                     preferred_element_type=jnp.float32)


def _attn_kernel(q_hbm, kt_ref, v_ref, seg_ref, o_hbm,
                 qbuf, obuf, in_sems, out_sems):
    def in_copy(i, slot):
        return pltpu.make_async_copy(
            q_hbm.at[pl.ds(i * CHUNK, CHUNK), :], qbuf.at[slot],
            in_sems.at[slot])

    def out_copy(i, slot):
        return pltpu.make_async_copy(
            obuf.at[slot], o_hbm.at[pl.ds(i * CHUNK, CHUNK), :],
            out_sems.at[slot])

    for s in range(NBUF):
        in_copy(s, s).start()

    def loop(i, carry):
        slot = jax.lax.rem(i, NBUF)
        in_copy(i, slot).wait()

        @pl.when(i >= NBUF)
        def _():
            out_copy(i - NBUF, slot).wait()

        _compute_chunk(qbuf.at[slot], kt_ref, v_ref, seg_ref, obuf.at[slot])
        out_copy(i, slot).start()

        @pl.when(i + NBUF < N_CHUNKS)
        def _():
            in_copy(i + NBUF, slot).start()

        return carry

    jax.lax.fori_loop(0, N_CHUNKS, loop, 0)
    for s in range(NBUF):
        i = N_CHUNKS - NBUF + s
        out_copy(i, i % NBUF).wait()


@jax.jit
def kernel(query, keys_0, values_0, salience_0, keys_1, values_1, salience_1,
           keys_2, values_2, salience_2):
    B, T, D = query.shape
    q2 = query.reshape(B * T, D)
    kt = (jnp.concatenate([keys_0, keys_1, keys_2], axis=0).T
          * INV_SQRT_D).astype(jnp.bfloat16)
    w = jnp.exp(jnp.concatenate([salience_0, salience_1, salience_2]))
    v = (jnp.concatenate([values_0, values_1, values_2], axis=0)
         * (w[:, None] * LEVEL_W)).astype(jnp.bfloat16)
    lvl = jnp.arange(S_TOTAL) // SEG
    seg = (lvl[:, None] == lvl[None, :]).astype(jnp.float32) * w[:, None]
    out = pl.pallas_call(
        _attn_kernel,
        in_specs=[
            pl.BlockSpec(memory_space=pltpu.MemorySpace.HBM),
            pl.BlockSpec(memory_space=pltpu.MemorySpace.VMEM),
            pl.BlockSpec(memory_space=pltpu.MemorySpace.VMEM),
            pl.BlockSpec(memory_space=pltpu.MemorySpace.VMEM),
        ],
        out_specs=pl.BlockSpec(memory_space=pltpu.MemorySpace.HBM),
        out_shape=jax.ShapeDtypeStruct((B * T, D), jnp.float32),
        scratch_shapes=[
            pltpu.VMEM((NBUF, CHUNK, D_MODEL), jnp.float32),
            pltpu.VMEM((NBUF, CHUNK, D_MODEL), jnp.float32),
            pltpu.SemaphoreType.DMA((NBUF,)),
            pltpu.SemaphoreType.DMA((NBUF,)),
        ],
    )(q2, kt, v, seg)
    return out.reshape(B, T, D)


# speedup vs baseline: 1.2176x; 1.0252x over previous
import math

import jax
import jax.numpy as jnp
from jax.experimental import pallas as pl
from jax.experimental.pallas import tpu as pltpu

D_MODEL = 2048
NUM_LVL = 3
SEG = 16
S_TOTAL = NUM_LVL * SEG
INV_SQRT_D = 1.0 / math.sqrt(D_MODEL)
LEVEL_W = 1.0 / NUM_LVL
CHUNK = 512
N_CHUNKS = 32
NBUF = 6
HALVES = 2


def _compute_chunk(qbuf_slot, kt_ref, v_ref, seg_ref, obuf_slot):
    h = CHUNK // HALVES
    for p in range(HALVES):
        rows = pl.ds(p * h, h)
        q = qbuf_slot[rows, :].astype(jnp.bfloat16)
        s = jnp.dot(q, kt_ref[...], preferred_element_type=jnp.float32)
        e = jnp.exp(s)
        z = jax.lax.dot_general(
            e, seg_ref[...], (((1,), (0,)), ((), ())),
            precision=jax.lax.Precision.HIGHEST,
            preferred_element_type=jnp.float32,
        )
        a = (e * (1.0 / z)).astype(jnp.bfloat16)
        obuf_slot[rows, :] = jnp.dot(a, v_ref[...],
                                     preferred_element_type=jnp.float32)


def _attn_kernel(q_hbm, kt_ref, v_ref, seg_ref, o_hbm,
                 qbuf, obuf, in_sems, out_sems):
    def in_copy(i, slot):
        return pltpu.make_async_copy(
            q_hbm.at[pl.ds(i * CHUNK, CHUNK), :], qbuf.at[slot],
            in_sems.at[slot])

    def out_copy(i, slot):
        return pltpu.make_async_copy(
            obuf.at[slot], o_hbm.at[pl.ds(i * CHUNK, CHUNK), :],
            out_sems.at[slot])

    for s in range(NBUF):
        in_copy(s, s).start()

    def loop(i, carry):
        slot = jax.lax.rem(i, NBUF)
        in_copy(i, slot).wait()

        @pl.when(i >= NBUF)
        def _():
            out_copy(i - NBUF, slot).wait()

        _compute_chunk(qbuf.at[slot], kt_ref, v_ref, seg_ref, obuf.at[slot])
        out_copy(i, slot).start()

        @pl.when(i + NBUF < N_CHUNKS)
        def _():
            in_copy(i + NBUF, slot).start()

        return carry

    jax.lax.fori_loop(0, N_CHUNKS, loop, 0)
    for s in range(NBUF):
        i = N_CHUNKS - NBUF + s
        out_copy(i, i % NBUF).wait()


@jax.jit
def kernel(query, keys_0, values_0, salience_0, keys_1, values_1, salience_1,
           keys_2, values_2, salience_2):
    B, T, D = query.shape
    q2 = query.reshape(B * T, D)
    kt = (jnp.concatenate([keys_0, keys_1, keys_2], axis=0).T
          * INV_SQRT_D).astype(jnp.bfloat16)
    w = jnp.exp(jnp.concatenate([salience_0, salience_1, salience_2]))
    v = (jnp.concatenate([values_0, values_1, values_2], axis=0)
         * (w[:, None] * LEVEL_W)).astype(jnp.bfloat16)
    lvl = jnp.arange(S_TOTAL) // SEG
    seg = (lvl[:, None] == lvl[None, :]).astype(jnp.float32) * w[:, None]
    out = pl.pallas_call(
        _attn_kernel,
        in_specs=[
            pl.BlockSpec(memory_space=pltpu.MemorySpace.HBM),
            pl.BlockSpec(memory_space=pltpu.MemorySpace.VMEM),
            pl.BlockSpec(memory_space=pltpu.MemorySpace.VMEM),
            pl.BlockSpec(memory_space=pltpu.MemorySpace.VMEM),
        ],
        out_specs=pl.BlockSpec(memory_space=pltpu.MemorySpace.HBM),
        out_shape=jax.ShapeDtypeStruct((B * T, D), jnp.float32),
        scratch_shapes=[
            pltpu.VMEM((NBUF, CHUNK, D_MODEL), jnp.float32),
            pltpu.VMEM((NBUF, CHUNK, D_MODEL), jnp.float32),
            pltpu.SemaphoreType.DMA((NBUF,)),
            pltpu.SemaphoreType.DMA((NBUF,)),
        ],
    )(q2, kt, v, seg)
    return out.reshape(B, T, D)
